# baseline (speedup 1.0000x reference)
import dataclasses
import functools

import jax
import jax.numpy as jnp
from jax import lax
from jax.experimental import pallas as pl
from jax.experimental.pallas import tpu as pltpu
from jax.experimental.pallas import tpu_sc as plsc

N_NODES = 10000
N_EDGES = 320000
IN_DIM = 128
HID_DIM = 128
OUT_DIM = 64
PRIOR_MU = 0.0
PRIOR_SIGMA = 0.1

NC = 2
NS = 16
NW = NC * NS
CHUNK = 128
NROWS = 2560
NROWS_ALLOC = 2608
E_PAD = NROWS_ALLOC * CHUNK
NPAD = 10112
DROWS = NPAD // 128
ROWS_PER_TILE = NPAD // NS
K0_DEG = 128
K1_DEG = 32
K0_AGG = 160
K1_AGG = 0
AGG_STAGE = 32

_sc_mesh = functools.partial(
    plsc.VectorSubcoreMesh, core_axis_name="c", subcore_axis_name="s")


def _sc_params():
  cp = pltpu.CompilerParams()
  if "needs_layout_passes" in pltpu.CompilerParams.__dataclass_fields__:
    cp = dataclasses.replace(cp, needs_layout_passes=False)
  return cp


def _deg_pallas(src2, dst2):
  out_type = (jax.ShapeDtypeStruct((NW, DROWS, 128), jnp.float32),
              jax.ShapeDtypeStruct((NW, DROWS, 128), jnp.float32))

  @functools.partial(
      pl.kernel, out_type=out_type, mesh=_sc_mesh(),
      compiler_params=_sc_params(),
      scratch_types=[
          pltpu.VMEM((K0_DEG, CHUNK), jnp.int32),
          pltpu.VMEM((K0_DEG, CHUNK), jnp.int32),
          pltpu.VMEM((DROWS, 128), jnp.float32),
          pltpu.VMEM((DROWS, 128), jnp.float32),
      ])
  def deg_kernel(src_hbm, dst_hbm, outs_hbm, outd_hbm,
                 sidx, didx, accs, accd):
    c = lax.axis_index("c")
    s = lax.axis_index("s")
    wid = c * NS + s
    nch_c = jnp.where(c == 0, K0_DEG, K1_DEG)
    base_c = pl.multiple_of(
        jnp.where(c == 0, s * K0_DEG, NS * K0_DEG + s * K1_DEG), 8)
    zero = jnp.zeros((16,), jnp.float32)

    @pl.loop(0, DROWS)
    def _(r):
      @pl.loop(0, 128, step=16)
      def _(k):
        accs[r, pl.ds(k, 16)] = zero
        accd[r, pl.ds(k, 16)] = zero

    pltpu.sync_copy(src_hbm.at[pl.ds(base_c, K0_DEG)], sidx)
    pltpu.sync_copy(dst_hbm.at[pl.ds(base_c, K0_DEG)], didx)
    ones = jnp.full((16,), 1.0, jnp.float32)

    @pl.loop(0, nch_c)
    def _(j):
      @pl.loop(0, CHUNK, step=16)
      def _(k):
        iv = sidx[j, pl.ds(k, 16)]
        plsc.addupdate_scatter(
            accs, [jnp.right_shift(iv, 7), jnp.bitwise_and(iv, 127)], ones)
        jv = didx[j, pl.ds(k, 16)]
        plsc.addupdate_scatter(
            accd, [jnp.right_shift(jv, 7), jnp.bitwise_and(jv, 127)], ones)

    pltpu.sync_copy(accs, outs_hbm.at[wid])
    pltpu.sync_copy(accd, outd_hbm.at[wid])

  return deg_kernel(src2, dst2)


def _agg_pallas(src2, dst2, hh, zrows):
  out_type = jax.ShapeDtypeStruct((NPAD, IN_DIM), jnp.float32)

  @functools.partial(
      pl.kernel, out_type=out_type, mesh=_sc_mesh(),
      scratch_types=[
          pltpu.VMEM((AGG_STAGE, CHUNK), jnp.int32),
          pltpu.VMEM((AGG_STAGE, CHUNK), jnp.int32),
          pltpu.VMEM((CHUNK, IN_DIM), jnp.float32),
          pltpu.VMEM((CHUNK, IN_DIM), jnp.float32),
          pltpu.VMEM_SHARED((NPAD, IN_DIM), jnp.float32),
          pltpu.SemaphoreType.DMA,
          pltpu.SemaphoreType.DMA,
      ])
  def agg_kernel(src_hbm, dst_hbm, hh_hbm, zrows_hbm, out_hbm,
                 sidx, didx, rows0, rows1, acc, sem0, sem1):
    c = lax.axis_index("c")
    s = lax.axis_index("s")
    r0 = s * ROWS_PER_TILE
    nch_c = jnp.where(c == 0, K0_AGG, K1_AGG)
    base_c = s * K0_AGG

    @pl.when(c == 0)
    def _():
      for z in range(5):
        pltpu.sync_copy(zrows_hbm, acc.at[pl.ds(r0 + z * 120, 120)])
      pltpu.sync_copy(zrows_hbm.at[pl.ds(0, 32)], acc.at[pl.ds(r0 + 600, 32)])

    plsc.subcore_barrier()

    for st in range(K0_AGG // AGG_STAGE):
      @pl.when(st * AGG_STAGE < nch_c)
      def _():
        row0 = pl.multiple_of(base_c + st * AGG_STAGE, 8)
        pltpu.sync_copy(src_hbm.at[pl.ds(row0, AGG_STAGE)], sidx)
        pltpu.sync_copy(dst_hbm.at[pl.ds(row0, AGG_STAGE)], didx)
        pltpu.async_copy(hh_hbm.at[sidx.at[0]], rows0, sem0)

        @pl.loop(0, AGG_STAGE, step=2)
        def _(j):
          pltpu.async_copy(hh_hbm.at[sidx.at[j + 1]], rows1, sem1)
          pltpu.make_async_copy(hh_hbm.at[sidx.at[j]], rows0, sem0).wait()
          pltpu.sync_copy(rows0, acc.at[didx.at[j]], add=True)

          @pl.when(j + 2 < AGG_STAGE)
          def _():
            pltpu.async_copy(hh_hbm.at[sidx.at[j + 2]], rows0, sem0)

          pltpu.make_async_copy(hh_hbm.at[sidx.at[j + 1]], rows1, sem1).wait()
          pltpu.sync_copy(rows1, acc.at[didx.at[j + 1]], add=True)

    plsc.subcore_barrier()

    @pl.when(c == 0)
    def _():
      pltpu.sync_copy(acc.at[pl.ds(r0, ROWS_PER_TILE)],
                      out_hbm.at[pl.ds(r0, ROWS_PER_TILE), :])

  return agg_kernel(src2, dst2, hh, zrows)


def _prep_pallas(xp3, degs_p, degd_p, ident):
  def body(x_ref, ds_ref, dd_ref, id_ref, hh_ref, xsum_ref, ncold_ref):
    deg_s = jnp.sum(ds_ref[...], axis=0)
    deg_d = jnp.sum(dd_ref[...], axis=0)
    norm_s = lax.rsqrt(jnp.maximum(deg_s, 1.0))
    norm_d = lax.rsqrt(jnp.maximum(deg_d, 1.0))
    dn = (((1,), (1,)), ((), ()))
    colmat_s = lax.dot_general(id_ref[...], norm_s, dn,
                               preferred_element_type=jnp.float32)
    colmat_d = lax.dot_general(id_ref[...], norm_d, dn,
                               preferred_element_type=jnp.float32)
    xb = x_ref[...]
    for r in range(DROWS):
      hh_ref[r] = xb[r] * colmat_s[:, r:r + 1]
      ncold_ref[r] = colmat_d[:, r:r + 1]
    s1 = jnp.sum(xb, axis=0)
    xsum_ref[...] = jnp.sum(s1, axis=0, keepdims=True)

  return pl.pallas_call(
      body,
      out_shape=[
          jax.ShapeDtypeStruct((DROWS, 128, IN_DIM), jnp.float32),
          jax.ShapeDtypeStruct((1, IN_DIM), jnp.float32),
          jax.ShapeDtypeStruct((DROWS, 128, 1), jnp.float32),
      ],
  )(xp3, degs_p, degd_p, ident)


def _softplus(rho):
  return jnp.log1p(jnp.exp(rho))


def _weights_pallas(c0_w_mu, c0_w_rho, c0_w_eps, c0_b_mu, c0_b_rho, c0_b_eps,
                    c1_w_mu, c1_w_rho, c1_b_mu, c1_b_rho,
                    p0_w_mu, p0_w_rho, p0_w_eps, p0_b_mu, p0_b_rho, p0_b_eps,
                    p1_w_mu, p1_w_rho, p1_w_eps, p1_b_mu, p1_b_rho, p1_b_eps):
  def kl_sum(mu, sigma):
    return jnp.sum(jnp.log(PRIOR_SIGMA / sigma)
                   + (sigma ** 2 + (mu - PRIOR_MU) ** 2)
                   / (2.0 * PRIOR_SIGMA ** 2) - 0.5)

  def body(c0wm, c0wr, c0we, c0bm, c0br, c0be,
           c1wm, c1wr, c1bm, c1br,
           p0wm, p0wr, p0we, p0bm, p0br, p0be,
           p1wm, p1wr, p1we, p1bm, p1br, p1be,
           w0_ref, b0_ref, wp0_ref, bp0_ref, wp1_ref, bp1_ref, kl_ref):
    c0ws = _softplus(c0wr[...])
    c0bs = _softplus(c0br[...])
    w0_ref[...] = c0wm[...] + c0ws * c0we[...]
    b0_ref[...] = c0bm[...] + c0bs * c0be[...]
    wp0_ref[...] = p0wm[...] + _softplus(p0wr[...]) * p0we[...]
    bp0_ref[...] = p0bm[...] + _softplus(p0br[...]) * p0be[...]
    wp1_ref[...] = p1wm[...] + _softplus(p1wr[...]) * p1we[...]
    bp1_ref[...] = p1bm[...] + _softplus(p1br[...]) * p1be[...]
    kl = (kl_sum(c0wm[...], c0ws) + kl_sum(c0bm[...], c0bs)
          + kl_sum(c1wm[...], _softplus(c1wr[...]))
          + kl_sum(c1bm[...], _softplus(c1br[...])))
    kl_ref[...] = jnp.reshape(kl, (1, 1))

  return pl.pallas_call(
      body,
      out_shape=[
          jax.ShapeDtypeStruct((IN_DIM, HID_DIM), jnp.float32),
          jax.ShapeDtypeStruct((1, HID_DIM), jnp.float32),
          jax.ShapeDtypeStruct((IN_DIM, OUT_DIM), jnp.float32),
          jax.ShapeDtypeStruct((1, OUT_DIM), jnp.float32),
          jax.ShapeDtypeStruct((HID_DIM, OUT_DIM), jnp.float32),
          jax.ShapeDtypeStruct((1, OUT_DIM), jnp.float32),
          jax.ShapeDtypeStruct((1, 1), jnp.float32),
      ],
  )(c0_w_mu, c0_w_rho, c0_w_eps, c0_b_mu, c0_b_rho, c0_b_eps,
    c1_w_mu, c1_w_rho, c1_b_mu, c1_b_rho,
    p0_w_mu, p0_w_rho, p0_w_eps, p0_b_mu, p0_b_rho, p0_b_eps,
    p1_w_mu, p1_w_rho, p1_w_eps, p1_b_mu, p1_b_rho, p1_b_eps)


_FIN_BLK = 2000
_FIN_GRID = N_NODES // _FIN_BLK


def _final_pallas(part, ncold, w0, b0, wp0, bp0, wp1, bp1, xsum):
  def body(p_ref, nd_ref, w0_ref, b0_ref, wp0_ref, bp0_ref, wp1_ref, bp1_ref,
           xsum_ref, out_ref, hsum_scr):
    i = pl.program_id(0)
    aggn = p_ref[...] * nd_ref[...]
    h = jnp.dot(aggn, w0_ref[...], preferred_element_type=jnp.float32)
    h = jnp.maximum(h + b0_ref[...], 0.0)

    @pl.when(i == 0)
    def _():
      hsum_scr[...] = jnp.zeros_like(hsum_scr)

    hsum_scr[...] += jnp.sum(h, axis=0, keepdims=True)

    @pl.when(i == _FIN_GRID - 1)
    def _():
      xm = xsum_ref[...] * (1.0 / N_NODES)
      hm = hsum_scr[...] * (1.0 / N_NODES)
      p0v = jnp.dot(xm, wp0_ref[...],
                    preferred_element_type=jnp.float32) + bp0_ref[...]
      p1v = jnp.dot(hm, wp1_ref[...],
                    preferred_element_type=jnp.float32) + bp1_ref[...]
      out_ref[...] = 0.5 * (p0v + p1v)

  return pl.pallas_call(
      body,
      grid=(_FIN_GRID,),
      in_specs=[
          pl.BlockSpec((_FIN_BLK, IN_DIM), lambda i: (i, 0)),
          pl.BlockSpec((_FIN_BLK, 1), lambda i: (i, 0)),
          pl.BlockSpec((IN_DIM, HID_DIM), lambda i: (0, 0)),
          pl.BlockSpec((1, HID_DIM), lambda i: (0, 0)),
          pl.BlockSpec((IN_DIM, OUT_DIM), lambda i: (0, 0)),
          pl.BlockSpec((1, OUT_DIM), lambda i: (0, 0)),
          pl.BlockSpec((HID_DIM, OUT_DIM), lambda i: (0, 0)),
          pl.BlockSpec((1, OUT_DIM), lambda i: (0, 0)),
          pl.BlockSpec((1, IN_DIM), lambda i: (0, 0)),
      ],
      out_specs=pl.BlockSpec((1, OUT_DIM), lambda i: (0, 0)),
      out_shape=jax.ShapeDtypeStruct((1, OUT_DIM), jnp.float32),
      scratch_shapes=[pltpu.VMEM((1, HID_DIM), jnp.float32)],
  )(part, ncold, w0, b0, wp0, bp0, wp1, bp1, xsum)


def kernel(x, edge_index,
           c0_w_mu, c0_w_rho, c0_w_eps, c0_b_mu, c0_b_rho, c0_b_eps,
           c1_w_mu, c1_w_rho, c1_w_eps, c1_b_mu, c1_b_rho, c1_b_eps,
           p0_w_mu, p0_w_rho, p0_w_eps, p0_b_mu, p0_b_rho, p0_b_eps,
           p1_w_mu, p1_w_rho, p1_w_eps, p1_b_mu, p1_b_rho, p1_b_eps):
  pad = jnp.full((E_PAD - N_EDGES,), N_NODES, jnp.int32)
  src2 = jnp.concatenate([edge_index[0], pad]).reshape(NROWS_ALLOC, CHUNK)
  dst2 = jnp.concatenate([edge_index[1], pad]).reshape(NROWS_ALLOC, CHUNK)
  xp3 = jnp.pad(x, ((0, NPAD - N_NODES), (0, 0))).reshape(DROWS, 128, IN_DIM)
  ident = jnp.eye(128, dtype=jnp.float32)
  zrows = jnp.zeros((120, IN_DIM), jnp.float32)

  degs_p, degd_p = _deg_pallas(src2, dst2)
  hh3, xsum, ncold3 = _prep_pallas(xp3, degs_p, degd_p, ident)
  hh = hh3.reshape(NPAD, IN_DIM)
  ncold = ncold3.reshape(NPAD, 1)
  w0, b0, wp0, bp0, wp1, bp1, kl2 = _weights_pallas(
      c0_w_mu, c0_w_rho, c0_w_eps,
      c0_b_mu.reshape(1, HID_DIM), c0_b_rho.reshape(1, HID_DIM),
      c0_b_eps.reshape(1, HID_DIM),
      c1_w_mu, c1_w_rho,
      c1_b_mu.reshape(1, HID_DIM), c1_b_rho.reshape(1, HID_DIM),
      p0_w_mu, p0_w_rho, p0_w_eps,
      p0_b_mu.reshape(1, OUT_DIM), p0_b_rho.reshape(1, OUT_DIM),
      p0_b_eps.reshape(1, OUT_DIM),
      p1_w_mu, p1_w_rho, p1_w_eps,
      p1_b_mu.reshape(1, OUT_DIM), p1_b_rho.reshape(1, OUT_DIM),
      p1_b_eps.reshape(1, OUT_DIM))
  part = _agg_pallas(src2, dst2, hh, zrows)
  out = _final_pallas(part, ncold, w0, b0, wp0, bp0, wp1, bp1, xsum)
  return (out, kl2[0, 0])


# speedup vs baseline: 1.0059x; 1.0059x over previous
import dataclasses
import functools

import jax
import jax.numpy as jnp
from jax import lax
from jax.experimental import pallas as pl
from jax.experimental.pallas import tpu as pltpu
from jax.experimental.pallas import tpu_sc as plsc

N_NODES = 10000
N_EDGES = 320000
IN_DIM = 128
HID_DIM = 128
OUT_DIM = 64
PRIOR_MU = 0.0
PRIOR_SIGMA = 0.1

NC = 2
NS = 16
NW = NC * NS
CHUNK = 128
NROWS = 2560
NROWS_ALLOC = 2608
E_PAD = NROWS_ALLOC * CHUNK
NPAD = 10112
DROWS = NPAD // 128
ROWS_PER_TILE = NPAD // NS
K0_DEG = 128
K1_DEG = 32
K0_AGG = 160
K1_AGG = 0
AGG_STAGE = 40

_sc_mesh = functools.partial(
    plsc.VectorSubcoreMesh, core_axis_name="c", subcore_axis_name="s")


def _sc_params():
  cp = pltpu.CompilerParams()
  if "needs_layout_passes" in pltpu.CompilerParams.__dataclass_fields__:
    cp = dataclasses.replace(cp, needs_layout_passes=False)
  return cp


def _deg_pallas(src2, dst2):
  out_type = (jax.ShapeDtypeStruct((NW, DROWS, 128), jnp.float32),
              jax.ShapeDtypeStruct((NW, DROWS, 128), jnp.float32))

  @functools.partial(
      pl.kernel, out_type=out_type, mesh=_sc_mesh(),
      compiler_params=_sc_params(),
      scratch_types=[
          pltpu.VMEM((K0_DEG, CHUNK), jnp.int32),
          pltpu.VMEM((K0_DEG, CHUNK), jnp.int32),
          pltpu.VMEM((DROWS, 128), jnp.float32),
          pltpu.VMEM((DROWS, 128), jnp.float32),
      ])
  def deg_kernel(src_hbm, dst_hbm, outs_hbm, outd_hbm,
                 sidx, didx, accs, accd):
    c = lax.axis_index("c")
    s = lax.axis_index("s")
    wid = c * NS + s
    nch_c = jnp.where(c == 0, K0_DEG, K1_DEG)
    base_c = pl.multiple_of(
        jnp.where(c == 0, s * K0_DEG, NS * K0_DEG + s * K1_DEG), 8)
    zero = jnp.zeros((16,), jnp.float32)

    @pl.loop(0, DROWS)
    def _(r):
      @pl.loop(0, 128, step=16)
      def _(k):
        accs[r, pl.ds(k, 16)] = zero
        accd[r, pl.ds(k, 16)] = zero

    pltpu.sync_copy(src_hbm.at[pl.ds(base_c, K0_DEG)], sidx)
    pltpu.sync_copy(dst_hbm.at[pl.ds(base_c, K0_DEG)], didx)
    ones = jnp.full((16,), 1.0, jnp.float32)

    @pl.loop(0, nch_c)
    def _(j):
      @pl.loop(0, CHUNK, step=16)
      def _(k):
        iv = sidx[j, pl.ds(k, 16)]
        plsc.addupdate_scatter(
            accs, [jnp.right_shift(iv, 7), jnp.bitwise_and(iv, 127)], ones)
        jv = didx[j, pl.ds(k, 16)]
        plsc.addupdate_scatter(
            accd, [jnp.right_shift(jv, 7), jnp.bitwise_and(jv, 127)], ones)

    pltpu.sync_copy(accs, outs_hbm.at[wid])
    pltpu.sync_copy(accd, outd_hbm.at[wid])

  return deg_kernel(src2, dst2)


def _agg_pallas(src2, dst2, hh, zrows):
  out_type = jax.ShapeDtypeStruct((NPAD, IN_DIM), jnp.float32)

  @functools.partial(
      pl.kernel, out_type=out_type, mesh=_sc_mesh(),
      scratch_types=[
          pltpu.VMEM((AGG_STAGE, CHUNK), jnp.int32),
          pltpu.VMEM((AGG_STAGE, CHUNK), jnp.int32),
          pltpu.VMEM((CHUNK, IN_DIM), jnp.float32),
          pltpu.VMEM((CHUNK, IN_DIM), jnp.float32),
          pltpu.VMEM_SHARED((NPAD, IN_DIM), jnp.float32),
          pltpu.SemaphoreType.DMA,
          pltpu.SemaphoreType.DMA,
      ])
  def agg_kernel(src_hbm, dst_hbm, hh_hbm, zrows_hbm, out_hbm,
                 sidx, didx, rows0, rows1, acc, sem0, sem1):
    c = lax.axis_index("c")
    s = lax.axis_index("s")
    r0 = s * ROWS_PER_TILE
    nch_c = jnp.where(c == 0, K0_AGG, K1_AGG)
    base_c = s * K0_AGG

    @pl.when(c == 0)
    def _():
      for z in range(5):
        pltpu.sync_copy(zrows_hbm, acc.at[pl.ds(r0 + z * 120, 120)])
      pltpu.sync_copy(zrows_hbm.at[pl.ds(0, 32)], acc.at[pl.ds(r0 + 600, 32)])

    plsc.subcore_barrier()

    for st in range(K0_AGG // AGG_STAGE):
      @pl.when(st * AGG_STAGE < nch_c)
      def _():
        row0 = pl.multiple_of(base_c + st * AGG_STAGE, 8)
        pltpu.sync_copy(src_hbm.at[pl.ds(row0, AGG_STAGE)], sidx)
        pltpu.sync_copy(dst_hbm.at[pl.ds(row0, AGG_STAGE)], didx)
        pltpu.async_copy(hh_hbm.at[sidx.at[0]], rows0, sem0)

        @pl.loop(0, AGG_STAGE, step=2)
        def _(j):
          pltpu.async_copy(hh_hbm.at[sidx.at[j + 1]], rows1, sem1)
          pltpu.make_async_copy(hh_hbm.at[sidx.at[j]], rows0, sem0).wait()
          pltpu.sync_copy(rows0, acc.at[didx.at[j]], add=True)

          @pl.when(j + 2 < AGG_STAGE)
          def _():
            pltpu.async_copy(hh_hbm.at[sidx.at[j + 2]], rows0, sem0)

          pltpu.make_async_copy(hh_hbm.at[sidx.at[j + 1]], rows1, sem1).wait()
          pltpu.sync_copy(rows1, acc.at[didx.at[j + 1]], add=True)

    plsc.subcore_barrier()

    @pl.when(c == 0)
    def _():
      pltpu.sync_copy(acc.at[pl.ds(r0, ROWS_PER_TILE)],
                      out_hbm.at[pl.ds(r0, ROWS_PER_TILE), :])

  return agg_kernel(src2, dst2, hh, zrows)


def _prep_pallas(xp3, degs_p, degd_p, ident):
  def body(x_ref, ds_ref, dd_ref, id_ref, hh_ref, xsum_ref, ncold_ref):
    deg_s = jnp.sum(ds_ref[...], axis=0)
    deg_d = jnp.sum(dd_ref[...], axis=0)
    norm_s = lax.rsqrt(jnp.maximum(deg_s, 1.0))
    norm_d = lax.rsqrt(jnp.maximum(deg_d, 1.0))
    dn = (((1,), (1,)), ((), ()))
    colmat_s = lax.dot_general(id_ref[...], norm_s, dn,
                               preferred_element_type=jnp.float32)
    colmat_d = lax.dot_general(id_ref[...], norm_d, dn,
                               preferred_element_type=jnp.float32)
    xb = x_ref[...]
    for r in range(DROWS):
      hh_ref[r] = xb[r] * colmat_s[:, r:r + 1]
      ncold_ref[r] = colmat_d[:, r:r + 1]
    s1 = jnp.sum(xb, axis=0)
    xsum_ref[...] = jnp.sum(s1, axis=0, keepdims=True)

  return pl.pallas_call(
      body,
      out_shape=[
          jax.ShapeDtypeStruct((DROWS, 128, IN_DIM), jnp.float32),
          jax.ShapeDtypeStruct((1, IN_DIM), jnp.float32),
          jax.ShapeDtypeStruct((DROWS, 128, 1), jnp.float32),
      ],
  )(xp3, degs_p, degd_p, ident)


def _softplus(rho):
  return jnp.log1p(jnp.exp(rho))


def _weights_pallas(c0_w_mu, c0_w_rho, c0_w_eps, c0_b_mu, c0_b_rho, c0_b_eps,
                    c1_w_mu, c1_w_rho, c1_b_mu, c1_b_rho,
                    p0_w_mu, p0_w_rho, p0_w_eps, p0_b_mu, p0_b_rho, p0_b_eps,
                    p1_w_mu, p1_w_rho, p1_w_eps, p1_b_mu, p1_b_rho, p1_b_eps):
  def kl_sum(mu, sigma):
    return jnp.sum(jnp.log(PRIOR_SIGMA / sigma)
                   + (sigma ** 2 + (mu - PRIOR_MU) ** 2)
                   / (2.0 * PRIOR_SIGMA ** 2) - 0.5)

  def body(c0wm, c0wr, c0we, c0bm, c0br, c0be,
           c1wm, c1wr, c1bm, c1br,
           p0wm, p0wr, p0we, p0bm, p0br, p0be,
           p1wm, p1wr, p1we, p1bm, p1br, p1be,
           w0_ref, b0_ref, wp0_ref, bp0_ref, wp1_ref, bp1_ref, kl_ref):
    c0ws = _softplus(c0wr[...])
    c0bs = _softplus(c0br[...])
    w0_ref[...] = c0wm[...] + c0ws * c0we[...]
    b0_ref[...] = c0bm[...] + c0bs * c0be[...]
    wp0_ref[...] = p0wm[...] + _softplus(p0wr[...]) * p0we[...]
    bp0_ref[...] = p0bm[...] + _softplus(p0br[...]) * p0be[...]
    wp1_ref[...] = p1wm[...] + _softplus(p1wr[...]) * p1we[...]
    bp1_ref[...] = p1bm[...] + _softplus(p1br[...]) * p1be[...]
    kl = (kl_sum(c0wm[...], c0ws) + kl_sum(c0bm[...], c0bs)
          + kl_sum(c1wm[...], _softplus(c1wr[...]))
          + kl_sum(c1bm[...], _softplus(c1br[...])))
    kl_ref[...] = jnp.reshape(kl, (1, 1))

  return pl.pallas_call(
      body,
      out_shape=[
          jax.ShapeDtypeStruct((IN_DIM, HID_DIM), jnp.float32),
          jax.ShapeDtypeStruct((1, HID_DIM), jnp.float32),
          jax.ShapeDtypeStruct((IN_DIM, OUT_DIM), jnp.float32),
          jax.ShapeDtypeStruct((1, OUT_DIM), jnp.float32),
          jax.ShapeDtypeStruct((HID_DIM, OUT_DIM), jnp.float32),
          jax.ShapeDtypeStruct((1, OUT_DIM), jnp.float32),
          jax.ShapeDtypeStruct((1, 1), jnp.float32),
      ],
  )(c0_w_mu, c0_w_rho, c0_w_eps, c0_b_mu, c0_b_rho, c0_b_eps,
    c1_w_mu, c1_w_rho, c1_b_mu, c1_b_rho,
    p0_w_mu, p0_w_rho, p0_w_eps, p0_b_mu, p0_b_rho, p0_b_eps,
    p1_w_mu, p1_w_rho, p1_w_eps, p1_b_mu, p1_b_rho, p1_b_eps)


_FIN_BLK = 2000
_FIN_GRID = N_NODES // _FIN_BLK


def _final_pallas(part, ncold, w0, b0, wp0, bp0, wp1, bp1, xsum):
  def body(p_ref, nd_ref, w0_ref, b0_ref, wp0_ref, bp0_ref, wp1_ref, bp1_ref,
           xsum_ref, out_ref, hsum_scr):
    i = pl.program_id(0)
    aggn = p_ref[...] * nd_ref[...]
    h = jnp.dot(aggn, w0_ref[...], preferred_element_type=jnp.float32)
    h = jnp.maximum(h + b0_ref[...], 0.0)

    @pl.when(i == 0)
    def _():
      hsum_scr[...] = jnp.zeros_like(hsum_scr)

    hsum_scr[...] += jnp.sum(h, axis=0, keepdims=True)

    @pl.when(i == _FIN_GRID - 1)
    def _():
      xm = xsum_ref[...] * (1.0 / N_NODES)
      hm = hsum_scr[...] * (1.0 / N_NODES)
      p0v = jnp.dot(xm, wp0_ref[...],
                    preferred_element_type=jnp.float32) + bp0_ref[...]
      p1v = jnp.dot(hm, wp1_ref[...],
                    preferred_element_type=jnp.float32) + bp1_ref[...]
      out_ref[...] = 0.5 * (p0v + p1v)

  return pl.pallas_call(
      body,
      grid=(_FIN_GRID,),
      in_specs=[
          pl.BlockSpec((_FIN_BLK, IN_DIM), lambda i: (i, 0)),
          pl.BlockSpec((_FIN_BLK, 1), lambda i: (i, 0)),
          pl.BlockSpec((IN_DIM, HID_DIM), lambda i: (0, 0)),
          pl.BlockSpec((1, HID_DIM), lambda i: (0, 0)),
          pl.BlockSpec((IN_DIM, OUT_DIM), lambda i: (0, 0)),
          pl.BlockSpec((1, OUT_DIM), lambda i: (0, 0)),
          pl.BlockSpec((HID_DIM, OUT_DIM), lambda i: (0, 0)),
          pl.BlockSpec((1, OUT_DIM), lambda i: (0, 0)),
          pl.BlockSpec((1, IN_DIM), lambda i: (0, 0)),
      ],
      out_specs=pl.BlockSpec((1, OUT_DIM), lambda i: (0, 0)),
      out_shape=jax.ShapeDtypeStruct((1, OUT_DIM), jnp.float32),
      scratch_shapes=[pltpu.VMEM((1, HID_DIM), jnp.float32)],
  )(part, ncold, w0, b0, wp0, bp0, wp1, bp1, xsum)


def kernel(x, edge_index,
           c0_w_mu, c0_w_rho, c0_w_eps, c0_b_mu, c0_b_rho, c0_b_eps,
           c1_w_mu, c1_w_rho, c1_w_eps, c1_b_mu, c1_b_rho, c1_b_eps,
           p0_w_mu, p0_w_rho, p0_w_eps, p0_b_mu, p0_b_rho, p0_b_eps,
           p1_w_mu, p1_w_rho, p1_w_eps, p1_b_mu, p1_b_rho, p1_b_eps):
  pad = jnp.full((E_PAD - N_EDGES,), N_NODES, jnp.int32)
  src2 = jnp.concatenate([edge_index[0], pad]).reshape(NROWS_ALLOC, CHUNK)
  dst2 = jnp.concatenate([edge_index[1], pad]).reshape(NROWS_ALLOC, CHUNK)
  xp3 = jnp.pad(x, ((0, NPAD - N_NODES), (0, 0))).reshape(DROWS, 128, IN_DIM)
  ident = jnp.eye(128, dtype=jnp.float32)
  zrows = jnp.zeros((120, IN_DIM), jnp.float32)

  degs_p, degd_p = _deg_pallas(src2, dst2)
  hh3, xsum, ncold3 = _prep_pallas(xp3, degs_p, degd_p, ident)
  hh = hh3.reshape(NPAD, IN_DIM)
  ncold = ncold3.reshape(NPAD, 1)
  w0, b0, wp0, bp0, wp1, bp1, kl2 = _weights_pallas(
      c0_w_mu, c0_w_rho, c0_w_eps,
      c0_b_mu.reshape(1, HID_DIM), c0_b_rho.reshape(1, HID_DIM),
      c0_b_eps.reshape(1, HID_DIM),
      c1_w_mu, c1_w_rho,
      c1_b_mu.reshape(1, HID_DIM), c1_b_rho.reshape(1, HID_DIM),
      p0_w_mu, p0_w_rho, p0_w_eps,
      p0_b_mu.reshape(1, OUT_DIM), p0_b_rho.reshape(1, OUT_DIM),
      p0_b_eps.reshape(1, OUT_DIM),
      p1_w_mu, p1_w_rho, p1_w_eps,
      p1_b_mu.reshape(1, OUT_DIM), p1_b_rho.reshape(1, OUT_DIM),
      p1_b_eps.reshape(1, OUT_DIM))
  part = _agg_pallas(src2, dst2, hh, zrows)
  out = _final_pallas(part, ncold, w0, b0, wp0, bp0, wp1, bp1, xsum)
  return (out, kl2[0, 0])


# speedup vs baseline: 1.2293x; 1.2222x over previous
import dataclasses
import functools

import jax
import jax.numpy as jnp
from jax import lax
from jax.experimental import pallas as pl
from jax.experimental.pallas import tpu as pltpu
from jax.experimental.pallas import tpu_sc as plsc

N_NODES = 10000
N_EDGES = 320000
IN_DIM = 128
HID_DIM = 128
OUT_DIM = 64
PRIOR_MU = 0.0
PRIOR_SIGMA = 0.1

NC = 2
NS = 16
NW = NC * NS
CHUNK = 128
NROWS = 2560
NROWS_ALLOC = 2608
E_PAD = NROWS_ALLOC * CHUNK
NPAD = 10112
DROWS = NPAD // 128
ROWS_PER_TILE = NPAD // NS
K0_DEG = 128
K1_DEG = 32
K0_AGG = 128
K1_AGG = 32
AGG_STAGE = 32

_sc_mesh = functools.partial(
    plsc.VectorSubcoreMesh, core_axis_name="c", subcore_axis_name="s")


def _sc_params():
  cp = pltpu.CompilerParams()
  if "needs_layout_passes" in pltpu.CompilerParams.__dataclass_fields__:
    cp = dataclasses.replace(cp, needs_layout_passes=False)
  return cp


def _deg_pallas(src2, dst2):
  out_type = (jax.ShapeDtypeStruct((NW, DROWS, 128), jnp.float32),
              jax.ShapeDtypeStruct((NW, DROWS, 128), jnp.float32))

  @functools.partial(
      pl.kernel, out_type=out_type, mesh=_sc_mesh(),
      compiler_params=_sc_params(),
      scratch_types=[
          pltpu.VMEM((K0_DEG, CHUNK), jnp.int32),
          pltpu.VMEM((K0_DEG, CHUNK), jnp.int32),
          pltpu.VMEM((DROWS, 128), jnp.float32),
          pltpu.VMEM((DROWS, 128), jnp.float32),
      ])
  def deg_kernel(src_hbm, dst_hbm, outs_hbm, outd_hbm,
                 sidx, didx, accs, accd):
    c = lax.axis_index("c")
    s = lax.axis_index("s")
    wid = c * NS + s
    nch_c = jnp.where(c == 0, K0_DEG, K1_DEG)
    base_c = pl.multiple_of(
        jnp.where(c == 0, s * K0_DEG, NS * K0_DEG + s * K1_DEG), 8)
    zero = jnp.zeros((16,), jnp.float32)

    @pl.loop(0, DROWS)
    def _(r):
      @pl.loop(0, 128, step=16)
      def _(k):
        accs[r, pl.ds(k, 16)] = zero
        accd[r, pl.ds(k, 16)] = zero

    pltpu.sync_copy(src_hbm.at[pl.ds(base_c, K0_DEG)], sidx)
    pltpu.sync_copy(dst_hbm.at[pl.ds(base_c, K0_DEG)], didx)
    ones = jnp.full((16,), 1.0, jnp.float32)

    @pl.loop(0, nch_c)
    def _(j):
      @pl.loop(0, CHUNK, step=16)
      def _(k):
        iv = sidx[j, pl.ds(k, 16)]
        plsc.addupdate_scatter(
            accs, [jnp.right_shift(iv, 7), jnp.bitwise_and(iv, 127)], ones)
        jv = didx[j, pl.ds(k, 16)]
        plsc.addupdate_scatter(
            accd, [jnp.right_shift(jv, 7), jnp.bitwise_and(jv, 127)], ones)

    pltpu.sync_copy(accs, outs_hbm.at[wid])
    pltpu.sync_copy(accd, outd_hbm.at[wid])

  return deg_kernel(src2, dst2)


def _agg_pallas(src2, dst2, hh, zrows):
  out_type = jax.ShapeDtypeStruct((NC, NPAD, IN_DIM), jnp.float32)

  @functools.partial(
      pl.kernel, out_type=out_type, mesh=_sc_mesh(),
      scratch_types=[
          pltpu.VMEM((AGG_STAGE, CHUNK), jnp.int32),
          pltpu.VMEM((AGG_STAGE, CHUNK), jnp.int32),
          pltpu.VMEM((CHUNK, IN_DIM), jnp.float32),
          pltpu.VMEM((CHUNK, IN_DIM), jnp.float32),
          pltpu.VMEM_SHARED((NPAD, IN_DIM), jnp.float32),
          pltpu.SemaphoreType.DMA,
          pltpu.SemaphoreType.DMA,
      ])
  def agg_kernel(src_hbm, dst_hbm, hh_hbm, zrows_hbm, out_hbm,
                 sidx, didx, rows0, rows1, acc, sem0, sem1):
    c = lax.axis_index("c")
    s = lax.axis_index("s")
    r0 = s * ROWS_PER_TILE
    nch_c = jnp.where(c == 0, K0_AGG, K1_AGG)
    base_c = jnp.where(c == 0, s * K0_AGG, NS * K0_AGG + s * K1_AGG)
    for z in range(5):
      pltpu.sync_copy(zrows_hbm, acc.at[pl.ds(r0 + z * 120, 120)])
    pltpu.sync_copy(zrows_hbm.at[pl.ds(0, 32)], acc.at[pl.ds(r0 + 600, 32)])
    plsc.subcore_barrier()

    for st in range(K0_AGG // AGG_STAGE):
      @pl.when(st * AGG_STAGE < nch_c)
      def _():
        row0 = pl.multiple_of(base_c + st * AGG_STAGE, 8)
        pltpu.sync_copy(src_hbm.at[pl.ds(row0, AGG_STAGE)], sidx)
        pltpu.sync_copy(dst_hbm.at[pl.ds(row0, AGG_STAGE)], didx)
        pltpu.async_copy(hh_hbm.at[sidx.at[0]], rows0, sem0)

        @pl.loop(0, AGG_STAGE, step=2)
        def _(j):
          pltpu.async_copy(hh_hbm.at[sidx.at[j + 1]], rows1, sem1)
          pltpu.make_async_copy(hh_hbm.at[sidx.at[j]], rows0, sem0).wait()
          pltpu.sync_copy(rows0, acc.at[didx.at[j]], add=True)

          @pl.when(j + 2 < AGG_STAGE)
          def _():
            pltpu.async_copy(hh_hbm.at[sidx.at[j + 2]], rows0, sem0)

          pltpu.make_async_copy(hh_hbm.at[sidx.at[j + 1]], rows1, sem1).wait()
          pltpu.sync_copy(rows1, acc.at[didx.at[j + 1]], add=True)

    plsc.subcore_barrier()
    pltpu.sync_copy(acc.at[pl.ds(r0, ROWS_PER_TILE)],
                    out_hbm.at[c, pl.ds(r0, ROWS_PER_TILE), :])

  return agg_kernel(src2, dst2, hh, zrows)


def _prep_pallas(xp3, degs_p, degd_p, ident):
  def body(x_ref, ds_ref, dd_ref, id_ref, hh_ref, xsum_ref, ncold_ref):
    deg_s = jnp.sum(ds_ref[...], axis=0)
    deg_d = jnp.sum(dd_ref[...], axis=0)
    norm_s = lax.rsqrt(jnp.maximum(deg_s, 1.0))
    norm_d = lax.rsqrt(jnp.maximum(deg_d, 1.0))
    dn = (((1,), (1,)), ((), ()))
    colmat_s = lax.dot_general(id_ref[...], norm_s, dn,
                               preferred_element_type=jnp.float32)
    colmat_d = lax.dot_general(id_ref[...], norm_d, dn,
                               preferred_element_type=jnp.float32)
    xb = x_ref[...]
    for r in range(DROWS):
      hh_ref[r] = xb[r] * colmat_s[:, r:r + 1]
      ncold_ref[r] = colmat_d[:, r:r + 1]
    s1 = jnp.sum(xb, axis=0)
    xsum_ref[...] = jnp.sum(s1, axis=0, keepdims=True)

  return pl.pallas_call(
      body,
      out_shape=[
          jax.ShapeDtypeStruct((DROWS, 128, IN_DIM), jnp.float32),
          jax.ShapeDtypeStruct((1, IN_DIM), jnp.float32),
          jax.ShapeDtypeStruct((DROWS, 128, 1), jnp.float32),
      ],
  )(xp3, degs_p, degd_p, ident)


def _softplus(rho):
  return jnp.log1p(jnp.exp(rho))


def _weights_pallas(c0_w_mu, c0_w_rho, c0_w_eps, c0_b_mu, c0_b_rho, c0_b_eps,
                    c1_w_mu, c1_w_rho, c1_b_mu, c1_b_rho,
                    p0_w_mu, p0_w_rho, p0_w_eps, p0_b_mu, p0_b_rho, p0_b_eps,
                    p1_w_mu, p1_w_rho, p1_w_eps, p1_b_mu, p1_b_rho, p1_b_eps):
  def kl_sum(mu, sigma):
    return jnp.sum(jnp.log(PRIOR_SIGMA / sigma)
                   + (sigma ** 2 + (mu - PRIOR_MU) ** 2)
                   / (2.0 * PRIOR_SIGMA ** 2) - 0.5)

  def body(c0wm, c0wr, c0we, c0bm, c0br, c0be,
           c1wm, c1wr, c1bm, c1br,
           p0wm, p0wr, p0we, p0bm, p0br, p0be,
           p1wm, p1wr, p1we, p1bm, p1br, p1be,
           w0_ref, b0_ref, wp0_ref, bp0_ref, wp1_ref, bp1_ref, kl_ref):
    c0ws = _softplus(c0wr[...])
    c0bs = _softplus(c0br[...])
    w0_ref[...] = c0wm[...] + c0ws * c0we[...]
    b0_ref[...] = c0bm[...] + c0bs * c0be[...]
    wp0_ref[...] = p0wm[...] + _softplus(p0wr[...]) * p0we[...]
    bp0_ref[...] = p0bm[...] + _softplus(p0br[...]) * p0be[...]
    wp1_ref[...] = p1wm[...] + _softplus(p1wr[...]) * p1we[...]
    bp1_ref[...] = p1bm[...] + _softplus(p1br[...]) * p1be[...]
    kl = (kl_sum(c0wm[...], c0ws) + kl_sum(c0bm[...], c0bs)
          + kl_sum(c1wm[...], _softplus(c1wr[...]))
          + kl_sum(c1bm[...], _softplus(c1br[...])))
    kl_ref[...] = jnp.reshape(kl, (1, 1))

  return pl.pallas_call(
      body,
      out_shape=[
          jax.ShapeDtypeStruct((IN_DIM, HID_DIM), jnp.float32),
          jax.ShapeDtypeStruct((1, HID_DIM), jnp.float32),
          jax.ShapeDtypeStruct((IN_DIM, OUT_DIM), jnp.float32),
          jax.ShapeDtypeStruct((1, OUT_DIM), jnp.float32),
          jax.ShapeDtypeStruct((HID_DIM, OUT_DIM), jnp.float32),
          jax.ShapeDtypeStruct((1, OUT_DIM), jnp.float32),
          jax.ShapeDtypeStruct((1, 1), jnp.float32),
      ],
  )(c0_w_mu, c0_w_rho, c0_w_eps, c0_b_mu, c0_b_rho, c0_b_eps,
    c1_w_mu, c1_w_rho, c1_b_mu, c1_b_rho,
    p0_w_mu, p0_w_rho, p0_w_eps, p0_b_mu, p0_b_rho, p0_b_eps,
    p1_w_mu, p1_w_rho, p1_w_eps, p1_b_mu, p1_b_rho, p1_b_eps)


_FIN_BLK = 2000
_FIN_GRID = N_NODES // _FIN_BLK


def _final_pallas(part, ncold, w0, b0, wp0, bp0, wp1, bp1, xsum):
  def body(p_ref, nd_ref, w0_ref, b0_ref, wp0_ref, bp0_ref, wp1_ref, bp1_ref,
           xsum_ref, out_ref, hsum_scr):
    i = pl.program_id(0)
    aggn = (p_ref[0] + p_ref[1]) * nd_ref[...]
    h = jnp.dot(aggn, w0_ref[...], preferred_element_type=jnp.float32)
    h = jnp.maximum(h + b0_ref[...], 0.0)

    @pl.when(i == 0)
    def _():
      hsum_scr[...] = jnp.zeros_like(hsum_scr)

    hsum_scr[...] += jnp.sum(h, axis=0, keepdims=True)

    @pl.when(i == _FIN_GRID - 1)
    def _():
      xm = xsum_ref[...] * (1.0 / N_NODES)
      hm = hsum_scr[...] * (1.0 / N_NODES)
      p0v = jnp.dot(xm, wp0_ref[...],
                    preferred_element_type=jnp.float32) + bp0_ref[...]
      p1v = jnp.dot(hm, wp1_ref[...],
                    preferred_element_type=jnp.float32) + bp1_ref[...]
      out_ref[...] = 0.5 * (p0v + p1v)

  return pl.pallas_call(
      body,
      grid=(_FIN_GRID,),
      in_specs=[
          pl.BlockSpec((2, _FIN_BLK, IN_DIM), lambda i: (0, i, 0)),
          pl.BlockSpec((_FIN_BLK, 1), lambda i: (i, 0)),
          pl.BlockSpec((IN_DIM, HID_DIM), lambda i: (0, 0)),
          pl.BlockSpec((1, HID_DIM), lambda i: (0, 0)),
          pl.BlockSpec((IN_DIM, OUT_DIM), lambda i: (0, 0)),
          pl.BlockSpec((1, OUT_DIM), lambda i: (0, 0)),
          pl.BlockSpec((HID_DIM, OUT_DIM), lambda i: (0, 0)),
          pl.BlockSpec((1, OUT_DIM), lambda i: (0, 0)),
          pl.BlockSpec((1, IN_DIM), lambda i: (0, 0)),
      ],
      out_specs=pl.BlockSpec((1, OUT_DIM), lambda i: (0, 0)),
      out_shape=jax.ShapeDtypeStruct((1, OUT_DIM), jnp.float32),
      scratch_shapes=[pltpu.VMEM((1, HID_DIM), jnp.float32)],
  )(part, ncold, w0, b0, wp0, bp0, wp1, bp1, xsum)


def kernel(x, edge_index,
           c0_w_mu, c0_w_rho, c0_w_eps, c0_b_mu, c0_b_rho, c0_b_eps,
           c1_w_mu, c1_w_rho, c1_w_eps, c1_b_mu, c1_b_rho, c1_b_eps,
           p0_w_mu, p0_w_rho, p0_w_eps, p0_b_mu, p0_b_rho, p0_b_eps,
           p1_w_mu, p1_w_rho, p1_w_eps, p1_b_mu, p1_b_rho, p1_b_eps):
  pad = jnp.full((E_PAD - N_EDGES,), N_NODES, jnp.int32)
  src2 = jnp.concatenate([edge_index[0], pad]).reshape(NROWS_ALLOC, CHUNK)
  dst2 = jnp.concatenate([edge_index[1], pad]).reshape(NROWS_ALLOC, CHUNK)
  xp3 = jnp.pad(x, ((0, NPAD - N_NODES), (0, 0))).reshape(DROWS, 128, IN_DIM)
  ident = jnp.eye(128, dtype=jnp.float32)
  zrows = jnp.zeros((120, IN_DIM), jnp.float32)

  degs_p, degd_p = _deg_pallas(src2, dst2)
  hh3, xsum, ncold3 = _prep_pallas(xp3, degs_p, degd_p, ident)
  hh = hh3.reshape(NPAD, IN_DIM)
  ncold = ncold3.reshape(NPAD, 1)
  w0, b0, wp0, bp0, wp1, bp1, kl2 = _weights_pallas(
      c0_w_mu, c0_w_rho, c0_w_eps,
      c0_b_mu.reshape(1, HID_DIM), c0_b_rho.reshape(1, HID_DIM),
      c0_b_eps.reshape(1, HID_DIM),
      c1_w_mu, c1_w_rho,
      c1_b_mu.reshape(1, HID_DIM), c1_b_rho.reshape(1, HID_DIM),
      p0_w_mu, p0_w_rho, p0_w_eps,
      p0_b_mu.reshape(1, OUT_DIM), p0_b_rho.reshape(1, OUT_DIM),
      p0_b_eps.reshape(1, OUT_DIM),
      p1_w_mu, p1_w_rho, p1_w_eps,
      p1_b_mu.reshape(1, OUT_DIM), p1_b_rho.reshape(1, OUT_DIM),
      p1_b_eps.reshape(1, OUT_DIM))
  part = _agg_pallas(src2, dst2, hh, zrows)
  out = _final_pallas(part, ncold, w0, b0, wp0, bp0, wp1, bp1, xsum)
  return (out, kl2[0, 0])
